# baseline (speedup 1.0000x reference)
import functools

import jax
import jax.numpy as jnp
from jax import lax
from jax.experimental import pallas as pl
from jax.experimental.pallas import tpu as pltpu
from jax.experimental.pallas import tpu_sc as plsc

N_NODES = 10000
N_EDGES = 320000
H = 128
VEC = 64
BN_EPS = 1e-5

NC = 2
NS = 16
NW = NC * NS
E_PAD = 327680
EPW = E_PAD // NW
CHUNK = 64
NCHUNK = EPW // CHUNK
NBUF = 2


def _pack_bf16_pair(lo, hi):
    def rnd(x):
        u = lax.bitcast_convert_type(x, jnp.uint32)
        return (u + jnp.uint32(0x7FFF) + ((u >> 16) & jnp.uint32(1))) >> 16
    return lax.bitcast_convert_type(rnd(lo) | (rnd(hi) << 16), jnp.int32)


def _node_proj_body(hv_ref, wn_ref, outs_ref, outd_ref):
    p = jnp.dot(hv_ref[...], wn_ref[...], preferred_element_type=jnp.float32)
    outs_ref[...] = p[:, :H]
    outd_ref[...] = p[:, H:]


def _unpack_bf16_pair(g):
    lo = lax.bitcast_convert_type(lax.shift_left(g, 16), jnp.float32)
    hi = lax.bitcast_convert_type(jnp.bitwise_and(g, jnp.int32(-65536)),
                                  jnp.float32)
    return jnp.concatenate([lo, hi], axis=1)


def _edge_mlp_body(g_ref, he_ref, vft_ref, we_ref, wv_ref, b_ref, out_ref):
    acc = jnp.dot(he_ref[...], we_ref[...], preferred_element_type=jnp.float32)
    acc = acc + lax.dot_general(vft_ref[...], wv_ref[...],
                                (((0,), (0,)), ((), ())),
                                preferred_element_type=jnp.float32)
    g = g_ref[...]
    even = lax.bitcast_convert_type(lax.shift_left(g, 16), jnp.float32)
    odd = lax.bitcast_convert_type(jnp.bitwise_and(g, jnp.int32(-65536)),
                                   jnp.float32)
    gi = jnp.stack([even, odd], axis=1).reshape(out_ref.shape)
    out_ref[...] = acc + gi + b_ref[...]


@functools.lru_cache(maxsize=1)
def _make_gather_sum():
    mesh = plsc.VectorSubcoreMesh(core_axis_name="c", subcore_axis_name="s")
    bf16 = jnp.bfloat16
    HP = H // 2

    @functools.partial(
        pl.kernel,
        mesh=mesh,
        out_type=jax.ShapeDtypeStruct((E_PAD // 2, H), jnp.int32),
        scratch_types=[
            pltpu.VMEM((NCHUNK, CHUNK), jnp.int32),
            pltpu.VMEM((NCHUNK, CHUNK), jnp.int32),
            pltpu.VMEM((NBUF, CHUNK, H), jnp.float32),
            pltpu.VMEM((NBUF, CHUNK, H), jnp.float32),
            pltpu.VMEM((NBUF, CHUNK // 2, H), jnp.int32),
        ] + [pltpu.SemaphoreType.DMA] * (2 * NBUF),
    )
    def gather_sum(ts_hbm, td_hbm, src_hbm, dst_hbm, out_hbm,
                   idxs_v, idxd_v, rows_s, rows_d, rows_o, *sems):
        wid = lax.axis_index("s") * NC + lax.axis_index("c")
        wbase = wid * (EPW // 2)
        gsems = sems[:NBUF]
        osems = sems[NBUF:]

        pltpu.sync_copy(src_hbm.at[wid], idxs_v)
        pltpu.sync_copy(dst_hbm.at[wid], idxd_v)

        def issue_gathers(j, b):
            pltpu.async_copy(ts_hbm.at[idxs_v.at[j]], rows_s.at[b], gsems[b])
            pltpu.async_copy(td_hbm.at[idxd_v.at[j]], rows_d.at[b], gsems[b])

        for b in range(NBUF):
            issue_gathers(b, b)

        def round_body(g, carry):
            for b in range(NBUF):
                j = NBUF * g + b
                pltpu.make_async_copy(ts_hbm.at[idxs_v.at[0]], rows_s.at[b],
                                      gsems[b]).wait()
                pltpu.make_async_copy(td_hbm.at[idxd_v.at[0]], rows_d.at[b],
                                      gsems[b]).wait()

                @pl.when(g > 0)
                def _():
                    pltpu.make_async_copy(
                        rows_o.at[b], out_hbm.at[pl.ds(wbase, CHUNK // 2)],
                        osems[b]).wait()

                def add_pair(p, c2):
                    re = 2 * p
                    ro = re + 1
                    for m in range(H // 16):
                        sl = pl.ds(m * 16, 16)
                        ae = rows_s[b, re, sl] + rows_d[b, re, sl]
                        ao = rows_s[b, ro, sl] + rows_d[b, ro, sl]
                        ue = lax.bitcast_convert_type(ae, jnp.uint32) + jnp.uint32(0x8000)
                        uo = lax.bitcast_convert_type(ao, jnp.uint32) + jnp.uint32(0x8000)
                        packed = (ue >> 16) | (uo & jnp.uint32(0xFFFF0000))
                        rows_o[b, p, sl] = lax.bitcast_convert_type(
                            packed, jnp.int32)
                    return c2

                lax.fori_loop(0, CHUNK // 2, add_pair, 0)
                obase = pl.multiple_of(wbase + j * (CHUNK // 2), 8)
                pltpu.async_copy(rows_o.at[b],
                                 out_hbm.at[pl.ds(obase, CHUNK // 2)],
                                 osems[b])

                @pl.when(j + NBUF < NCHUNK)
                def _():
                    issue_gathers(j + NBUF, b)
            return carry

        lax.fori_loop(0, NCHUNK // NBUF, round_body, 0)
        for b in range(NBUF):
            pltpu.make_async_copy(rows_o.at[b],
                                  out_hbm.at[pl.ds(wbase, CHUNK // 2)],
                                  osems[b]).wait()

    return gather_sum


def kernel(h_V, h_E, edge_idx, batch_id, vector_field_feat_to_s,
           W11_w, W11_b, bn_gamma, bn_beta, bn_mean, bn_var):
    f32 = jnp.float32
    scale = bn_gamma * lax.rsqrt(bn_var + BN_EPS)
    shift = bn_beta - bn_mean * scale
    Ws = (W11_w[:, :H] * scale[:, None]).T
    We = (W11_w[:, H:2 * H] * scale[:, None]).T + jnp.diag(scale)
    Wd = (W11_w[:, 2 * H:3 * H] * scale[:, None]).T
    Wv = (W11_w[:, 3 * H:] * scale[:, None]).T
    b_eff = (W11_b * scale + shift).reshape(1, H)
    Wn = jnp.concatenate([Ws, Wd], axis=1)

    table_s, table_d = pl.pallas_call(
        _node_proj_body,
        out_shape=[jax.ShapeDtypeStruct((N_NODES, H), jnp.float32),
                   jax.ShapeDtypeStruct((N_NODES, H), jnp.float32)],
    )(h_V, Wn)

    idx_pad = jnp.pad(edge_idx, ((0, 0), (0, E_PAD - N_EDGES)))
    src3 = idx_pad[0].reshape(NW, NCHUNK, CHUNK)
    dst3 = idx_pad[1].reshape(NW, NCHUNK, CHUNK)
    G = _make_gather_sum()(table_s, table_d, src3, dst3)

    BLK = 2560
    vft = vector_field_feat_to_s.T
    out = pl.pallas_call(
        _edge_mlp_body,
        grid=(N_EDGES // BLK,),
        in_specs=[
            pl.BlockSpec((BLK // 2, H), lambda i: (i, 0)),
            pl.BlockSpec((BLK, H), lambda i: (i, 0)),
            pl.BlockSpec((VEC, BLK), lambda i: (0, i)),
            pl.BlockSpec((H, H), lambda i: (0, 0)),
            pl.BlockSpec((VEC, H), lambda i: (0, 0)),
            pl.BlockSpec((1, H), lambda i: (0, 0)),
        ],
        out_specs=pl.BlockSpec((BLK, H), lambda i: (i, 0)),
        out_shape=jax.ShapeDtypeStruct((N_EDGES, H), f32),
    )(G, h_E, vft, We, Wv, b_eff)
    return out


# speedup vs baseline: 1.6809x; 1.6809x over previous
import functools

import jax
import jax.numpy as jnp
from jax import lax
from jax.experimental import pallas as pl
from jax.experimental.pallas import tpu as pltpu
from jax.experimental.pallas import tpu_sc as plsc

N_NODES = 10000
N_EDGES = 320000
H = 128
VEC = 64
BN_EPS = 1e-5

NC = 2
NS = 16
NW = NC * NS
EPW = N_EDGES // NW
CHUNK = 40
NCHUNK = EPW // CHUNK
NBUF = 2


def _pack_bf16_pair(lo, hi):
    def rnd(x):
        u = lax.bitcast_convert_type(x, jnp.uint32)
        return (u + jnp.uint32(0x7FFF) + ((u >> 16) & jnp.uint32(1))) >> 16
    return lax.bitcast_convert_type(rnd(lo) | (rnd(hi) << 16), jnp.int32)


def _node_proj_body(hv_ref, wn_ref, outs_ref, outd_ref):
    p = jnp.dot(hv_ref[...], wn_ref[...], preferred_element_type=jnp.float32)
    outs_ref[...] = _pack_bf16_pair(p[:, 0:64], p[:, 64:128])
    outd_ref[...] = _pack_bf16_pair(p[:, 128:192], p[:, 192:256])


def _unpack_bf16_pair(g):
    lo = lax.bitcast_convert_type(lax.shift_left(g, 16), jnp.float32)
    hi = lax.bitcast_convert_type(jnp.bitwise_and(g, jnp.int32(-65536)),
                                  jnp.float32)
    return jnp.concatenate([lo, hi], axis=1)


def _edge_mlp_body(g_ref, he_ref, vft_ref, we_ref, wv_ref, b_ref, out_ref):
    acc = jnp.dot(he_ref[...], we_ref[...], preferred_element_type=jnp.float32)
    acc = acc + lax.dot_general(vft_ref[...], wv_ref[...],
                                (((0,), (0,)), ((), ())),
                                preferred_element_type=jnp.float32)
    out_ref[...] = acc + _unpack_bf16_pair(g_ref[...]) + b_ref[...]


@functools.lru_cache(maxsize=1)
def _make_gather_sum():
    mesh = plsc.VectorSubcoreMesh(core_axis_name="c", subcore_axis_name="s")
    bf16 = jnp.bfloat16
    HP = H // 2

    @functools.partial(
        pl.kernel,
        mesh=mesh,
        out_type=jax.ShapeDtypeStruct((N_EDGES, HP), jnp.int32),
        scratch_types=[
            pltpu.VMEM((NCHUNK, CHUNK), jnp.int32),
            pltpu.VMEM((NCHUNK, CHUNK), jnp.int32),
            pltpu.VMEM((NBUF, CHUNK, HP), jnp.int32),
            pltpu.VMEM((NBUF, CHUNK, HP), jnp.int32),
            pltpu.VMEM((NBUF, CHUNK, HP), jnp.int32),
        ] + [pltpu.SemaphoreType.DMA] * (2 * NBUF),
        compiler_params=pltpu.CompilerParams(use_tc_tiling_on_sc=False),
    )
    def gather_sum(ts_hbm, td_hbm, src_hbm, dst_hbm, out_hbm,
                   idxs_v, idxd_v, rows_s, rows_d, rows_o, *sems):
        wid = lax.axis_index("s") * NC + lax.axis_index("c")
        wbase = wid * EPW
        gsems = sems[:NBUF]
        osems = sems[NBUF:]

        pltpu.sync_copy(src_hbm.at[wid], idxs_v)
        pltpu.sync_copy(dst_hbm.at[wid], idxd_v)

        def issue_gathers(j, b):
            pltpu.async_copy(ts_hbm.at[idxs_v.at[j]], rows_s.at[b], gsems[b])
            pltpu.async_copy(td_hbm.at[idxd_v.at[j]], rows_d.at[b], gsems[b])

        for b in range(NBUF):
            issue_gathers(b, b)

        def round_body(g, carry):
            for b in range(NBUF):
                j = NBUF * g + b
                pltpu.make_async_copy(ts_hbm.at[idxs_v.at[0]], rows_s.at[b],
                                      gsems[b]).wait()
                pltpu.make_async_copy(td_hbm.at[idxd_v.at[0]], rows_d.at[b],
                                      gsems[b]).wait()

                @pl.when(g > 0)
                def _():
                    pltpu.make_async_copy(
                        rows_o.at[b], out_hbm.at[pl.ds(wbase, CHUNK)],
                        osems[b]).wait()

                def add_row(r, c2):
                    mask = jnp.uint32(0xFFFF0000)
                    for m in range(HP // 16):
                        sl = pl.ds(m * 16, 16)
                        s = lax.bitcast_convert_type(rows_s[b, r, sl],
                                                     jnp.uint32)
                        d = lax.bitcast_convert_type(rows_d[b, r, sl],
                                                     jnp.uint32)
                        lo = (lax.bitcast_convert_type(s << 16, jnp.float32)
                              + lax.bitcast_convert_type(d << 16, jnp.float32))
                        hi = (lax.bitcast_convert_type(s & mask, jnp.float32)
                              + lax.bitcast_convert_type(d & mask, jnp.float32))
                        ulo = lax.bitcast_convert_type(lo, jnp.uint32) + jnp.uint32(0x8000)
                        uhi = lax.bitcast_convert_type(hi, jnp.uint32) + jnp.uint32(0x8000)
                        packed = (ulo >> 16) | (uhi & mask)
                        rows_o[b, r, sl] = lax.bitcast_convert_type(
                            packed, jnp.int32)
                    return c2

                lax.fori_loop(0, CHUNK, add_row, 0)
                obase = pl.multiple_of(wbase + j * CHUNK, 8)
                pltpu.async_copy(rows_o.at[b],
                                 out_hbm.at[pl.ds(obase, CHUNK)],
                                 osems[b])

                @pl.when(j + NBUF < NCHUNK)
                def _():
                    issue_gathers(j + NBUF, b)
            return carry

        lax.fori_loop(0, NCHUNK // NBUF, round_body, 0)
        for b in range(NBUF):
            pltpu.make_async_copy(rows_o.at[b],
                                  out_hbm.at[pl.ds(wbase, CHUNK)],
                                  osems[b]).wait()

    return gather_sum


def kernel(h_V, h_E, edge_idx, batch_id, vector_field_feat_to_s,
           W11_w, W11_b, bn_gamma, bn_beta, bn_mean, bn_var):
    f32 = jnp.float32
    scale = bn_gamma * lax.rsqrt(bn_var + BN_EPS)
    shift = bn_beta - bn_mean * scale
    Ws = (W11_w[:, :H] * scale[:, None]).T
    We = (W11_w[:, H:2 * H] * scale[:, None]).T + jnp.diag(scale)
    Wd = (W11_w[:, 2 * H:3 * H] * scale[:, None]).T
    Wv = (W11_w[:, 3 * H:] * scale[:, None]).T
    b_eff = (W11_b * scale + shift).reshape(1, H)
    Wn = jnp.concatenate([Ws, Wd], axis=1)

    table_s, table_d = pl.pallas_call(
        _node_proj_body,
        out_shape=[jax.ShapeDtypeStruct((N_NODES, H // 2), jnp.int32),
                   jax.ShapeDtypeStruct((N_NODES, H // 2), jnp.int32)],
    )(h_V, Wn)

    src3 = edge_idx[0].reshape(NW, NCHUNK, CHUNK)
    dst3 = edge_idx[1].reshape(NW, NCHUNK, CHUNK)
    G = _make_gather_sum()(table_s, table_d, src3, dst3)

    BLK = 2560
    vft = vector_field_feat_to_s.T
    out = pl.pallas_call(
        _edge_mlp_body,
        grid=(N_EDGES // BLK,),
        in_specs=[
            pl.BlockSpec((BLK, H // 2), lambda i: (i, 0)),
            pl.BlockSpec((BLK, H), lambda i: (i, 0)),
            pl.BlockSpec((VEC, BLK), lambda i: (0, i)),
            pl.BlockSpec((H, H), lambda i: (0, 0)),
            pl.BlockSpec((VEC, H), lambda i: (0, 0)),
            pl.BlockSpec((1, H), lambda i: (0, 0)),
        ],
        out_specs=pl.BlockSpec((BLK, H), lambda i: (i, 0)),
        out_shape=jax.ShapeDtypeStruct((N_EDGES, H), f32),
    )(G, h_E, vft, We, Wv, b_eff)
    return out


# speedup vs baseline: 2.0146x; 1.1985x over previous
import functools

import jax
import jax.numpy as jnp
from jax import lax
from jax.experimental import pallas as pl
from jax.experimental.pallas import tpu as pltpu
from jax.experimental.pallas import tpu_sc as plsc

N_NODES = 10000
N_EDGES = 320000
H = 128
VEC = 64
BN_EPS = 1e-5

NC = 2
NS = 16
NW = NC * NS
EPW = N_EDGES // NW
CHUNK = 40
NCHUNK = EPW // CHUNK
NBUF = 2


def _pack_bf16_pair(lo, hi):
    def rnd(x):
        u = lax.bitcast_convert_type(x, jnp.uint32)
        return (u + jnp.uint32(0x7FFF) + ((u >> 16) & jnp.uint32(1))) >> 16
    return lax.bitcast_convert_type(rnd(lo) | (rnd(hi) << 16), jnp.int32)


def _node_proj_body(hv_ref, wn_ref, outs_ref, outd_ref):
    p = jnp.dot(hv_ref[...], wn_ref[...], preferred_element_type=jnp.float32)
    outs_ref[...] = _pack_bf16_pair(p[:, 0:64], p[:, 64:128])
    outd_ref[...] = _pack_bf16_pair(p[:, 128:192], p[:, 192:256])


def _unpack_bf16_pair(g):
    lo = lax.bitcast_convert_type(lax.shift_left(g, 16), jnp.float32)
    hi = lax.bitcast_convert_type(jnp.bitwise_and(g, jnp.int32(-65536)),
                                  jnp.float32)
    return jnp.concatenate([lo, hi], axis=1)


def _edge_mlp_body(g_ref, he_ref, vft_ref, we_ref, wv_ref, b_ref, out_ref):
    acc = jnp.dot(he_ref[...], we_ref[...], preferred_element_type=jnp.float32)
    acc = acc + lax.dot_general(vft_ref[...], wv_ref[...],
                                (((0,), (0,)), ((), ())),
                                preferred_element_type=jnp.float32)
    g = g_ref[...]
    even = lax.bitcast_convert_type(lax.shift_left(g, 16), jnp.float32)
    odd = lax.bitcast_convert_type(jnp.bitwise_and(g, jnp.int32(-65536)),
                                   jnp.float32)
    gi = jnp.stack([even, odd], axis=1).reshape(out_ref.shape)
    out_ref[...] = acc + gi + b_ref[...]


@functools.lru_cache(maxsize=1)
def _make_gather_sum():
    mesh = plsc.VectorSubcoreMesh(core_axis_name="c", subcore_axis_name="s")
    bf16 = jnp.bfloat16
    HP = H // 2

    @functools.partial(
        pl.kernel,
        mesh=mesh,
        out_type=jax.ShapeDtypeStruct((N_EDGES // 2, H), jnp.int32),
        scratch_types=[
            pltpu.VMEM((NCHUNK, CHUNK), jnp.int32),
            pltpu.VMEM((NCHUNK, CHUNK), jnp.int32),
            pltpu.VMEM((NBUF, CHUNK, HP), jnp.int32),
            pltpu.VMEM((NBUF, CHUNK, HP), jnp.int32),
            pltpu.VMEM((NBUF, CHUNK // 2, H), jnp.int32),
        ] + [pltpu.SemaphoreType.DMA] * (2 * NBUF),
        compiler_params=pltpu.CompilerParams(use_tc_tiling_on_sc=False),
    )
    def gather_sum(ts_hbm, td_hbm, src_hbm, dst_hbm, out_hbm,
                   idxs_v, idxd_v, rows_s, rows_d, rows_o, *sems):
        wid = lax.axis_index("s") * NC + lax.axis_index("c")
        wbase = wid * (EPW // 2)
        gsems = sems[:NBUF]
        osems = sems[NBUF:]

        pltpu.sync_copy(src_hbm.at[wid], idxs_v)
        pltpu.sync_copy(dst_hbm.at[wid], idxd_v)

        def issue_gathers(j, b):
            pltpu.async_copy(ts_hbm.at[idxs_v.at[j]], rows_s.at[b], gsems[b])
            pltpu.async_copy(td_hbm.at[idxd_v.at[j]], rows_d.at[b], gsems[b])

        for b in range(NBUF):
            issue_gathers(b, b)

        def round_body(g, carry):
            for b in range(NBUF):
                j = NBUF * g + b
                pltpu.make_async_copy(ts_hbm.at[idxs_v.at[0]], rows_s.at[b],
                                      gsems[b]).wait()
                pltpu.make_async_copy(td_hbm.at[idxd_v.at[0]], rows_d.at[b],
                                      gsems[b]).wait()

                @pl.when(g > 0)
                def _():
                    pltpu.make_async_copy(
                        rows_o.at[b], out_hbm.at[pl.ds(wbase, CHUNK // 2)],
                        osems[b]).wait()

                def add_pair(p, c2):
                    mask = jnp.uint32(0xFFFF0000)
                    u32 = jnp.uint32
                    f32 = jnp.float32
                    bc = lax.bitcast_convert_type
                    re = 2 * p
                    ro = re + 1
                    for m in range(HP // 16):
                        sl = pl.ds(m * 16, 16)
                        a_s = bc(rows_s[b, re, sl], u32)
                        a_d = bc(rows_d[b, re, sl], u32)
                        b_s = bc(rows_s[b, ro, sl], u32)
                        b_d = bc(rows_d[b, ro, sl], u32)
                        alo = bc(a_s << 16, f32) + bc(a_d << 16, f32)
                        ahi = bc(a_s & mask, f32) + bc(a_d & mask, f32)
                        blo = bc(b_s << 16, f32) + bc(b_d << 16, f32)
                        bhi = bc(b_s & mask, f32) + bc(b_d & mask, f32)
                        ualo = bc(alo, u32) + u32(0x8000)
                        uahi = bc(ahi, u32) + u32(0x8000)
                        ublo = bc(blo, u32) + u32(0x8000)
                        ubhi = bc(bhi, u32) + u32(0x8000)
                        rows_o[b, p, sl] = bc((ualo >> 16) | (ublo & mask),
                                              jnp.int32)
                        rows_o[b, p, pl.ds(64 + m * 16, 16)] = bc(
                            (uahi >> 16) | (ubhi & mask), jnp.int32)
                    return c2

                lax.fori_loop(0, CHUNK // 2, add_pair, 0)
                obase = pl.multiple_of(wbase + j * (CHUNK // 2), 4)
                pltpu.async_copy(rows_o.at[b],
                                 out_hbm.at[pl.ds(obase, CHUNK // 2)],
                                 osems[b])

                @pl.when(j + NBUF < NCHUNK)
                def _():
                    issue_gathers(j + NBUF, b)
            return carry

        lax.fori_loop(0, NCHUNK // NBUF, round_body, 0)
        for b in range(NBUF):
            pltpu.make_async_copy(rows_o.at[b],
                                  out_hbm.at[pl.ds(wbase, CHUNK // 2)],
                                  osems[b]).wait()

    return gather_sum


def kernel(h_V, h_E, edge_idx, batch_id, vector_field_feat_to_s,
           W11_w, W11_b, bn_gamma, bn_beta, bn_mean, bn_var):
    f32 = jnp.float32
    scale = bn_gamma * lax.rsqrt(bn_var + BN_EPS)
    shift = bn_beta - bn_mean * scale
    Ws = (W11_w[:, :H] * scale[:, None]).T
    We = (W11_w[:, H:2 * H] * scale[:, None]).T + jnp.diag(scale)
    Wd = (W11_w[:, 2 * H:3 * H] * scale[:, None]).T
    Wv = (W11_w[:, 3 * H:] * scale[:, None]).T
    b_eff = (W11_b * scale + shift).reshape(1, H)
    Wn = jnp.concatenate([Ws, Wd], axis=1)

    table_s, table_d = pl.pallas_call(
        _node_proj_body,
        out_shape=[jax.ShapeDtypeStruct((N_NODES, H // 2), jnp.int32),
                   jax.ShapeDtypeStruct((N_NODES, H // 2), jnp.int32)],
    )(h_V, Wn)

    src3 = edge_idx[0].reshape(NW, NCHUNK, CHUNK)
    dst3 = edge_idx[1].reshape(NW, NCHUNK, CHUNK)
    G = _make_gather_sum()(table_s, table_d, src3, dst3)

    BLK = 2560
    vft = vector_field_feat_to_s.T
    out = pl.pallas_call(
        _edge_mlp_body,
        grid=(N_EDGES // BLK,),
        in_specs=[
            pl.BlockSpec((BLK // 2, H), lambda i: (i, 0)),
            pl.BlockSpec((BLK, H), lambda i: (i, 0)),
            pl.BlockSpec((VEC, BLK), lambda i: (0, i)),
            pl.BlockSpec((H, H), lambda i: (0, 0)),
            pl.BlockSpec((VEC, H), lambda i: (0, 0)),
            pl.BlockSpec((1, H), lambda i: (0, 0)),
        ],
        out_specs=pl.BlockSpec((BLK, H), lambda i: (i, 0)),
        out_shape=jax.ShapeDtypeStruct((N_EDGES, H), f32),
    )(G, h_E, vft, We, Wv, b_eff)
    return out


# speedup vs baseline: 2.7163x; 1.3483x over previous
import functools

import jax
import jax.numpy as jnp
from jax import lax
from jax.experimental import pallas as pl
from jax.experimental.pallas import tpu as pltpu
from jax.experimental.pallas import tpu_sc as plsc

N_NODES = 10000
N_EDGES = 320000
H = 128
VEC = 64
BN_EPS = 1e-5

NC = 2
NS = 16
NW = NC * NS
EPW = N_EDGES // NW
CHUNK = 40
NCHUNK = EPW // CHUNK
NBUF = 5


def _pack_bf16_pair(lo, hi):
    def rnd(x):
        u = lax.bitcast_convert_type(x, jnp.uint32)
        return (u + jnp.uint32(0x7FFF) + ((u >> 16) & jnp.uint32(1))) >> 16
    return lax.bitcast_convert_type(rnd(lo) | (rnd(hi) << 16), jnp.int32)


def _node_proj_body(hv_ref, wn_ref, outs_ref, outd_ref):
    p = jnp.dot(hv_ref[...], wn_ref[...], preferred_element_type=jnp.float32)
    outs_ref[...] = _pack_bf16_pair(p[:, 0:64], p[:, 64:128])
    outd_ref[...] = _pack_bf16_pair(p[:, 128:192], p[:, 192:256])


def _unpack_bf16_pair(g):
    lo = lax.bitcast_convert_type(lax.shift_left(g, 16), jnp.float32)
    hi = lax.bitcast_convert_type(jnp.bitwise_and(g, jnp.int32(-65536)),
                                  jnp.float32)
    return jnp.concatenate([lo, hi], axis=1)


def _edge_mlp_body(g_ref, he_ref, vft_ref, we_ref, wv_ref, b_ref, out_ref):
    acc = jnp.dot(he_ref[...], we_ref[...], preferred_element_type=jnp.float32)
    acc = acc + lax.dot_general(vft_ref[...], wv_ref[...],
                                (((0,), (0,)), ((), ())),
                                preferred_element_type=jnp.float32)
    g = g_ref[...]
    even = lax.bitcast_convert_type(lax.shift_left(g, 16), jnp.float32)
    odd = lax.bitcast_convert_type(jnp.bitwise_and(g, jnp.int32(-65536)),
                                   jnp.float32)
    gi = jnp.stack([even, odd], axis=1).reshape(out_ref.shape)
    out_ref[...] = acc + gi + b_ref[...]


@functools.lru_cache(maxsize=1)
def _make_gather_sum():
    mesh = plsc.VectorSubcoreMesh(core_axis_name="c", subcore_axis_name="s")
    bf16 = jnp.bfloat16
    HP = H // 2

    @functools.partial(
        pl.kernel,
        mesh=mesh,
        out_type=jax.ShapeDtypeStruct((N_EDGES // 2, H), jnp.int32),
        scratch_types=[
            pltpu.VMEM((NCHUNK, CHUNK), jnp.int32),
            pltpu.VMEM((NCHUNK, CHUNK), jnp.int32),
            pltpu.VMEM((NBUF, CHUNK, HP), jnp.int32),
            pltpu.VMEM((NBUF, CHUNK, HP), jnp.int32),
            pltpu.VMEM((NBUF, CHUNK // 2, H), jnp.int32),
        ] + [pltpu.SemaphoreType.DMA] * (2 * NBUF),
        compiler_params=pltpu.CompilerParams(use_tc_tiling_on_sc=False),
    )
    def gather_sum(ts_hbm, td_hbm, src_hbm, dst_hbm, out_hbm,
                   idxs_v, idxd_v, rows_s, rows_d, rows_o, *sems):
        wid = lax.axis_index("s") * NC + lax.axis_index("c")
        wbase = wid * (EPW // 2)
        gsems = sems[:NBUF]
        osems = sems[NBUF:]

        pltpu.sync_copy(src_hbm.at[wid], idxs_v)
        pltpu.sync_copy(dst_hbm.at[wid], idxd_v)

        def issue_gathers(j, b):
            pltpu.async_copy(ts_hbm.at[idxs_v.at[j]], rows_s.at[b], gsems[b])
            pltpu.async_copy(td_hbm.at[idxd_v.at[j]], rows_d.at[b], gsems[b])

        for b in range(NBUF):
            issue_gathers(b, b)

        def round_body(g, carry):
            for b in range(NBUF):
                j = NBUF * g + b
                pltpu.make_async_copy(ts_hbm.at[idxs_v.at[0]], rows_s.at[b],
                                      gsems[b]).wait()
                pltpu.make_async_copy(td_hbm.at[idxd_v.at[0]], rows_d.at[b],
                                      gsems[b]).wait()

                @pl.when(g > 0)
                def _():
                    pltpu.make_async_copy(
                        rows_o.at[b], out_hbm.at[pl.ds(wbase, CHUNK // 2)],
                        osems[b]).wait()

                def add_pair(p, c2):
                    mask = jnp.uint32(0xFFFF0000)
                    u32 = jnp.uint32
                    f32 = jnp.float32
                    bc = lax.bitcast_convert_type
                    re = 2 * p
                    ro = re + 1
                    for m in range(HP // 16):
                        sl = pl.ds(m * 16, 16)
                        a_s = bc(rows_s[b, re, sl], u32)
                        a_d = bc(rows_d[b, re, sl], u32)
                        b_s = bc(rows_s[b, ro, sl], u32)
                        b_d = bc(rows_d[b, ro, sl], u32)
                        alo = bc(a_s << 16, f32) + bc(a_d << 16, f32)
                        ahi = bc(a_s & mask, f32) + bc(a_d & mask, f32)
                        blo = bc(b_s << 16, f32) + bc(b_d << 16, f32)
                        bhi = bc(b_s & mask, f32) + bc(b_d & mask, f32)
                        ualo = bc(alo, u32) + u32(0x8000)
                        uahi = bc(ahi, u32) + u32(0x8000)
                        ublo = bc(blo, u32) + u32(0x8000)
                        ubhi = bc(bhi, u32) + u32(0x8000)
                        rows_o[b, p, sl] = bc((ualo >> 16) | (ublo & mask),
                                              jnp.int32)
                        rows_o[b, p, pl.ds(64 + m * 16, 16)] = bc(
                            (uahi >> 16) | (ubhi & mask), jnp.int32)
                    return c2

                lax.fori_loop(0, CHUNK // 2, add_pair, 0)
                obase = pl.multiple_of(wbase + j * (CHUNK // 2), 4)
                pltpu.async_copy(rows_o.at[b],
                                 out_hbm.at[pl.ds(obase, CHUNK // 2)],
                                 osems[b])

                @pl.when(j + NBUF < NCHUNK)
                def _():
                    issue_gathers(j + NBUF, b)
            return carry

        lax.fori_loop(0, NCHUNK // NBUF, round_body, 0)
        for b in range(NBUF):
            pltpu.make_async_copy(rows_o.at[b],
                                  out_hbm.at[pl.ds(wbase, CHUNK // 2)],
                                  osems[b]).wait()

    return gather_sum


def kernel(h_V, h_E, edge_idx, batch_id, vector_field_feat_to_s,
           W11_w, W11_b, bn_gamma, bn_beta, bn_mean, bn_var):
    f32 = jnp.float32
    scale = bn_gamma * lax.rsqrt(bn_var + BN_EPS)
    shift = bn_beta - bn_mean * scale
    Ws = (W11_w[:, :H] * scale[:, None]).T
    We = (W11_w[:, H:2 * H] * scale[:, None]).T + jnp.diag(scale)
    Wd = (W11_w[:, 2 * H:3 * H] * scale[:, None]).T
    Wv = (W11_w[:, 3 * H:] * scale[:, None]).T
    b_eff = (W11_b * scale + shift).reshape(1, H)
    Wn = jnp.concatenate([Ws, Wd], axis=1)

    table_s, table_d = pl.pallas_call(
        _node_proj_body,
        out_shape=[jax.ShapeDtypeStruct((N_NODES, H // 2), jnp.int32),
                   jax.ShapeDtypeStruct((N_NODES, H // 2), jnp.int32)],
    )(h_V, Wn)

    src3 = edge_idx[0].reshape(NW, NCHUNK, CHUNK)
    dst3 = edge_idx[1].reshape(NW, NCHUNK, CHUNK)
    G = _make_gather_sum()(table_s, table_d, src3, dst3)

    BLK = 6400
    vft = vector_field_feat_to_s.T
    out = pl.pallas_call(
        _edge_mlp_body,
        grid=(N_EDGES // BLK,),
        in_specs=[
            pl.BlockSpec((BLK // 2, H), lambda i: (i, 0)),
            pl.BlockSpec((BLK, H), lambda i: (i, 0)),
            pl.BlockSpec((VEC, BLK), lambda i: (0, i)),
            pl.BlockSpec((H, H), lambda i: (0, 0)),
            pl.BlockSpec((VEC, H), lambda i: (0, 0)),
            pl.BlockSpec((1, H), lambda i: (0, 0)),
        ],
        out_specs=pl.BlockSpec((BLK, H), lambda i: (i, 0)),
        out_shape=jax.ShapeDtypeStruct((N_EDGES, H), f32),
    )(G, h_E, vft, We, Wv, b_eff)
    return out


# speedup vs baseline: 2.8331x; 1.0430x over previous
import functools

import jax
import jax.numpy as jnp
from jax import lax
from jax.experimental import pallas as pl
from jax.experimental.pallas import tpu as pltpu
from jax.experimental.pallas import tpu_sc as plsc

N_NODES = 10000
N_EDGES = 320000
H = 128
VEC = 64
BN_EPS = 1e-5

NC = 2
NS = 16
NW = NC * NS
EPW = N_EDGES // NW
CHUNK = 40
NCHUNK = EPW // CHUNK
NBUF = 5


def _pack_bf16_pair(lo, hi):
    def rnd(x):
        u = lax.bitcast_convert_type(x, jnp.uint32)
        return (u + jnp.uint32(0x7FFF) + ((u >> 16) & jnp.uint32(1))) >> 16
    return lax.bitcast_convert_type(rnd(lo) | (rnd(hi) << 16), jnp.int32)


def _node_proj_body(hv_ref, wn_ref, outs_ref, outd_ref):
    p = jnp.dot(hv_ref[...], wn_ref[...], preferred_element_type=jnp.float32)
    outs_ref[...] = _pack_bf16_pair(p[:, 0:64], p[:, 64:128])
    outd_ref[...] = _pack_bf16_pair(p[:, 128:192], p[:, 192:256])


def _unpack_bf16_pair(g):
    lo = lax.bitcast_convert_type(lax.shift_left(g, 16), jnp.float32)
    hi = lax.bitcast_convert_type(jnp.bitwise_and(g, jnp.int32(-65536)),
                                  jnp.float32)
    return jnp.concatenate([lo, hi], axis=1)


def _edge_mlp_body(g_ref, he_ref, vft_ref, we_ref, wv_ref, b_ref, out_ref):
    acc = jnp.dot(he_ref[...], we_ref[...], preferred_element_type=jnp.float32)
    acc = acc + lax.dot_general(vft_ref[...], wv_ref[...],
                                (((0,), (0,)), ((), ())),
                                preferred_element_type=jnp.float32)
    g = g_ref[...]
    even = lax.bitcast_convert_type(lax.shift_left(g, 16), jnp.float32)
    odd = lax.bitcast_convert_type(jnp.bitwise_and(g, jnp.int32(-65536)),
                                   jnp.float32)
    gi = jnp.stack([even, odd], axis=1).reshape(out_ref.shape)
    out_ref[...] = acc + gi + b_ref[...]


@functools.lru_cache(maxsize=1)
def _make_gather_sum():
    mesh = plsc.VectorSubcoreMesh(core_axis_name="c", subcore_axis_name="s")
    bf16 = jnp.bfloat16
    HP = H // 2

    @functools.partial(
        pl.kernel,
        mesh=mesh,
        out_type=jax.ShapeDtypeStruct((N_EDGES // 2, H), jnp.int32),
        scratch_types=[
            pltpu.VMEM((NCHUNK, CHUNK), jnp.int32),
            pltpu.VMEM((NCHUNK, CHUNK), jnp.int32),
            pltpu.VMEM((NBUF, CHUNK, HP), jnp.int32),
            pltpu.VMEM((NBUF, CHUNK, HP), jnp.int32),
            pltpu.VMEM((NBUF, CHUNK // 2, H), jnp.int32),
        ] + [pltpu.SemaphoreType.DMA] * (2 * NBUF),
        compiler_params=pltpu.CompilerParams(use_tc_tiling_on_sc=False),
    )
    def gather_sum(ts_hbm, td_hbm, src_hbm, dst_hbm, out_hbm,
                   idxs_v, idxd_v, rows_s, rows_d, rows_o, *sems):
        wid = lax.axis_index("s") * NC + lax.axis_index("c")
        wbase = wid * (EPW // 2)
        gsems = sems[:NBUF]
        osems = sems[NBUF:]

        pltpu.sync_copy(src_hbm.at[wid], idxs_v)
        pltpu.sync_copy(dst_hbm.at[wid], idxd_v)

        def issue_gathers(j, b):
            pltpu.async_copy(ts_hbm.at[idxs_v.at[j]], rows_s.at[b], gsems[b])
            pltpu.async_copy(td_hbm.at[idxd_v.at[j]], rows_d.at[b], gsems[b])

        for b in range(NBUF):
            issue_gathers(b, b)

        def round_body(g, carry):
            for b in range(NBUF):
                j = NBUF * g + b
                pltpu.make_async_copy(ts_hbm.at[idxs_v.at[0]], rows_s.at[b],
                                      gsems[b]).wait()
                pltpu.make_async_copy(td_hbm.at[idxd_v.at[0]], rows_d.at[b],
                                      gsems[b]).wait()

                @pl.when(g > 0)
                def _():
                    pltpu.make_async_copy(
                        rows_o.at[b], out_hbm.at[pl.ds(wbase, CHUNK // 2)],
                        osems[b]).wait()

                def add_pair(p, c2):
                    mask = jnp.uint32(0xFFFF0000)
                    u32 = jnp.uint32
                    f32 = jnp.float32
                    bc = lax.bitcast_convert_type
                    re = 2 * p
                    ro = re + 1
                    for m in range(HP // 16):
                        sl = pl.ds(m * 16, 16)
                        a_s = bc(rows_s[b, re, sl], u32)
                        a_d = bc(rows_d[b, re, sl], u32)
                        b_s = bc(rows_s[b, ro, sl], u32)
                        b_d = bc(rows_d[b, ro, sl], u32)
                        alo = bc(a_s << 16, f32) + bc(a_d << 16, f32)
                        ahi = bc(a_s & mask, f32) + bc(a_d & mask, f32)
                        blo = bc(b_s << 16, f32) + bc(b_d << 16, f32)
                        bhi = bc(b_s & mask, f32) + bc(b_d & mask, f32)
                        ualo = bc(alo, u32) + u32(0x8000)
                        uahi = bc(ahi, u32) + u32(0x8000)
                        ublo = bc(blo, u32) + u32(0x8000)
                        ubhi = bc(bhi, u32) + u32(0x8000)
                        rows_o[b, p, sl] = bc((ualo >> 16) | (ublo & mask),
                                              jnp.int32)
                        rows_o[b, p, pl.ds(64 + m * 16, 16)] = bc(
                            (uahi >> 16) | (ubhi & mask), jnp.int32)
                    return c2

                lax.fori_loop(0, CHUNK // 2, add_pair, 0)
                obase = pl.multiple_of(wbase + j * (CHUNK // 2), 4)
                pltpu.async_copy(rows_o.at[b],
                                 out_hbm.at[pl.ds(obase, CHUNK // 2)],
                                 osems[b])

                @pl.when(j + NBUF < NCHUNK)
                def _():
                    issue_gathers(j + NBUF, b)
            return carry

        lax.fori_loop(0, NCHUNK // NBUF, round_body, 0)
        for b in range(NBUF):
            pltpu.make_async_copy(rows_o.at[b],
                                  out_hbm.at[pl.ds(wbase, CHUNK // 2)],
                                  osems[b]).wait()

    return gather_sum


def kernel(h_V, h_E, edge_idx, batch_id, vector_field_feat_to_s,
           W11_w, W11_b, bn_gamma, bn_beta, bn_mean, bn_var):
    f32 = jnp.float32
    scale = bn_gamma * lax.rsqrt(bn_var + BN_EPS)
    shift = bn_beta - bn_mean * scale
    Ws = (W11_w[:, :H] * scale[:, None]).T
    We = (W11_w[:, H:2 * H] * scale[:, None]).T + jnp.diag(scale)
    Wd = (W11_w[:, 2 * H:3 * H] * scale[:, None]).T
    Wv = (W11_w[:, 3 * H:] * scale[:, None]).T
    b_eff = (W11_b * scale + shift).reshape(1, H)
    Wn = jnp.concatenate([Ws, Wd], axis=1)

    table_s, table_d = pl.pallas_call(
        _node_proj_body,
        out_shape=[jax.ShapeDtypeStruct((N_NODES, H // 2), jnp.int32),
                   jax.ShapeDtypeStruct((N_NODES, H // 2), jnp.int32)],
    )(h_V, Wn)

    src3 = edge_idx[0].reshape(NW, NCHUNK, CHUNK)
    dst3 = edge_idx[1].reshape(NW, NCHUNK, CHUNK)
    G = _make_gather_sum()(table_s, table_d, src3, dst3)

    BLK = 12800
    vft = vector_field_feat_to_s.T
    out = pl.pallas_call(
        _edge_mlp_body,
        grid=(N_EDGES // BLK,),
        in_specs=[
            pl.BlockSpec((BLK // 2, H), lambda i: (i, 0)),
            pl.BlockSpec((BLK, H), lambda i: (i, 0)),
            pl.BlockSpec((VEC, BLK), lambda i: (0, i)),
            pl.BlockSpec((H, H), lambda i: (0, 0)),
            pl.BlockSpec((VEC, H), lambda i: (0, 0)),
            pl.BlockSpec((1, H), lambda i: (0, 0)),
        ],
        out_specs=pl.BlockSpec((BLK, H), lambda i: (i, 0)),
        out_shape=jax.ShapeDtypeStruct((N_EDGES, H), f32),
    )(G, h_E, vft, We, Wv, b_eff)
    return out
